# Initial kernel scaffold; baseline (speedup 1.0000x reference)
import jax
import jax.numpy as jnp
from jax.experimental import pallas as pl


def kernel(og_x, edge_index, edge_weight, W, att_l, att_r, W_res):
    raise NotImplementedError("write your pallas kernel here")


# speedup vs baseline: 31.4578x; 31.4578x over previous
import functools

import jax
import jax.numpy as jnp
from jax import lax
from jax.experimental import pallas as pl
from jax.experimental.pallas import tpu as pltpu
from jax.experimental.pallas import tpu_sc as plsc

N = 10000
E = 320000
D = 128
H = 8
C = 16
CH = 128
NCHUNK = E // CH


def _mm_body(ogx_ref, wt_ref, wrt_ref, prr_ref, x_ref, res_ref, ar_ref):
    xb = jnp.dot(ogx_ref[...], wt_ref[...], preferred_element_type=jnp.float32)
    x_ref[...] = xb
    res_ref[...] = jnp.dot(ogx_ref[...], wrt_ref[...],
                           preferred_element_type=jnp.float32)
    ar_ref[...] = jnp.dot(xb, prr_ref[...], preferred_element_type=jnp.float32)


def _tc_matmuls(og_x, wt, wrt, pr_rep):
    bm = 1000
    return pl.pallas_call(
        _mm_body,
        grid=(N // bm,),
        in_specs=[
            pl.BlockSpec((bm, D), lambda i: (i, 0)),
            pl.BlockSpec((D, D), lambda i: (0, 0)),
            pl.BlockSpec((D, D), lambda i: (0, 0)),
            pl.BlockSpec((D, D), lambda i: (0, 0)),
        ],
        out_specs=[
            pl.BlockSpec((bm, D), lambda i: (i, 0)),
            pl.BlockSpec((bm, D), lambda i: (i, 0)),
            pl.BlockSpec((bm, D), lambda i: (i, 0)),
        ],
        out_shape=[
            jax.ShapeDtypeStruct((N, D), jnp.float32),
            jax.ShapeDtypeStruct((N, D), jnp.float32),
            jax.ShapeDtypeStruct((N, D), jnp.float32),
        ],
    )(og_x, wt, wrt, pr_rep)


def _edge_body(xg_ref, arg_ref, ewb_ref, plr_ref, b8_ref, e_ref, emul_ref):
    alg = jnp.dot(xg_ref[...], plr_ref[...], preferred_element_type=jnp.float32)
    ew128 = jnp.dot(ewb_ref[...], b8_ref[...],
                    preferred_element_type=jnp.float32)
    z = ew128 * (alg + arg_ref[...])
    z = jnp.where(z >= 0.0, z, 0.2 * z)
    e128 = jnp.exp(z)
    e_ref[...] = e128
    emul_ref[...] = xg_ref[...] * e128


def _tc_edge(xg, arg, ewb, pl_rep, b8):
    bm = 2000
    return pl.pallas_call(
        _edge_body,
        grid=(E // bm,),
        in_specs=[
            pl.BlockSpec((bm, D), lambda i: (i, 0)),
            pl.BlockSpec((bm, D), lambda i: (i, 0)),
            pl.BlockSpec((bm, H), lambda i: (i, 0)),
            pl.BlockSpec((D, D), lambda i: (0, 0)),
            pl.BlockSpec((H, D), lambda i: (0, 0)),
        ],
        out_specs=[
            pl.BlockSpec((bm, D), lambda i: (i, 0)),
            pl.BlockSpec((bm, D), lambda i: (i, 0)),
        ],
        out_shape=[
            jax.ShapeDtypeStruct((E, D), jnp.float32),
            jax.ShapeDtypeStruct((E, D), jnp.float32),
        ],
    )(xg, arg, ewb, pl_rep, b8)


def _final_body(np_ref, dp_ref, res_ref, o_ref):
    den = jnp.sum(dp_ref[...], axis=0) + 1e-16
    o = jnp.sum(np_ref[...], axis=0) / den
    o = jnp.where(o > 0.0, o, jnp.exp(o) - 1.0)
    o_ref[...] = o + res_ref[...]


def _tc_final(numer_parts, denom_parts, res):
    ncp = numer_parts.shape[0]
    bm = 1000
    return pl.pallas_call(
        _final_body,
        grid=(N // bm,),
        in_specs=[
            pl.BlockSpec((ncp, bm, D), lambda i: (0, i, 0)),
            pl.BlockSpec((ncp, bm, D), lambda i: (0, i, 0)),
            pl.BlockSpec((bm, D), lambda i: (i, 0)),
        ],
        out_specs=pl.BlockSpec((bm, D), lambda i: (i, 0)),
        out_shape=jax.ShapeDtypeStruct((N, D), jnp.float32),
    )(numer_parts, denom_parts, res)


def _make_sc_gather(nc, ns):
    nw = nc * ns
    nloop = -(-NCHUNK // nw)
    mesh = plsc.VectorSubcoreMesh(core_axis_name="c", subcore_axis_name="s")

    @functools.partial(
        pl.kernel,
        mesh=mesh,
        out_type=[
            jax.ShapeDtypeStruct((E, D), jnp.float32),
            jax.ShapeDtypeStruct((E, D), jnp.float32),
        ],
        scratch_types=[
            pltpu.VMEM((CH,), jnp.int32),
            pltpu.VMEM((CH,), jnp.int32),
            pltpu.VMEM((CH, D), jnp.float32),
            pltpu.VMEM((CH, D), jnp.float32),
            pltpu.SemaphoreType.DMA,
        ],
    )
    def k(x_hbm, ar_hbm, src, dst, o_xg, o_arg, src_v, dst_v, xr_v, ar_v, sem):
        wid = lax.axis_index("s") * nc + lax.axis_index("c")

        def body(i, carry):
            cid = wid + i * nw

            @pl.when(cid < NCHUNK)
            def _():
                base = cid * CH
                pltpu.sync_copy(src.at[pl.ds(base, CH)], src_v)
                pltpu.sync_copy(dst.at[pl.ds(base, CH)], dst_v)
                pltpu.async_copy(x_hbm.at[src_v], xr_v, sem).wait()
                pltpu.async_copy(ar_hbm.at[dst_v], ar_v, sem).wait()
                pltpu.sync_copy(xr_v, o_xg.at[pl.ds(base, CH)])
                pltpu.sync_copy(ar_v, o_arg.at[pl.ds(base, CH)])

            return carry

        lax.fori_loop(0, nloop, body, 0)

    return k


def _make_sc_scatter(nc, ns):
    nw = nc * ns
    nloop = -(-NCHUNK // nw)
    mesh = plsc.VectorSubcoreMesh(core_axis_name="c", subcore_axis_name="s")

    @functools.partial(
        pl.kernel,
        mesh=mesh,
        out_type=[
            jax.ShapeDtypeStruct((nc, N, D), jnp.float32),
            jax.ShapeDtypeStruct((nc, N, D), jnp.float32),
        ],
        scratch_types=[
            pltpu.VMEM_SHARED((N, D), jnp.float32),
            pltpu.VMEM((CH,), jnp.int32),
            pltpu.VMEM((CH, D), jnp.float32),
        ],
    )
    def k(emul, e128, dst, zero_nd, o_num, o_den, shared, dst_v, val_v):
        cidx = lax.axis_index("c")
        sidx = lax.axis_index("s")
        wid = sidx * nc + cidx

        def scatter_phase(vals_hbm, out_hbm):
            @pl.when(sidx == 0)
            def _():
                pltpu.sync_copy(zero_nd, shared)

            plsc.subcore_barrier()

            def body(i, carry):
                cid = wid + i * nw

                @pl.when(cid < NCHUNK)
                def _():
                    base = cid * CH
                    pltpu.sync_copy(dst.at[pl.ds(base, CH)], dst_v)
                    pltpu.sync_copy(vals_hbm.at[pl.ds(base, CH)], val_v)
                    pltpu.sync_copy(val_v, shared.at[dst_v], add=True)

                return carry

            lax.fori_loop(0, nloop, body, 0)
            plsc.subcore_barrier()

            @pl.when(sidx == 0)
            def _():
                pltpu.sync_copy(shared, out_hbm.at[cidx])

            plsc.subcore_barrier()

        scatter_phase(emul, o_num)
        scatter_phase(e128, o_den)

    return k


def kernel(og_x, edge_index, edge_weight, W, att_l, att_r, W_res):
    src = edge_index[0]
    dst = edge_index[1]

    info = plsc.get_sparse_core_info()
    nc, ns = info.num_cores, info.num_subcores

    eye8 = jnp.eye(H, dtype=jnp.float32)
    pl_blk = (att_l[0][:, :, None] * eye8[:, None, :]).reshape(H * C, H)
    pr_blk = (att_r[0][:, :, None] * eye8[:, None, :]).reshape(H * C, H)
    b8 = jnp.repeat(eye8, C, axis=1)
    pl_rep = pl_blk @ b8
    pr_rep = pr_blk @ b8

    x, res, ar128 = _tc_matmuls(og_x, W.T, W_res.T, pr_rep)

    xg, arg = _make_sc_gather(nc, ns)(x, ar128, src, dst)

    ewb = jnp.repeat(edge_weight, H).reshape(E, H)
    e128, emul = _tc_edge(xg, arg, ewb, pl_rep, b8)

    zero_nd = jnp.zeros((N, D), jnp.float32)
    numer_parts, denom_parts = _make_sc_scatter(nc, ns)(
        emul, e128, dst, zero_nd)

    return _tc_final(numer_parts, denom_parts, res)


# speedup vs baseline: 33.2480x; 1.0569x over previous
import functools

import jax
import jax.numpy as jnp
from jax import lax
from jax.experimental import pallas as pl
from jax.experimental.pallas import tpu as pltpu
from jax.experimental.pallas import tpu_sc as plsc

N = 10000
E = 320000
D = 128
H = 8
C = 16
CH = 128
NCHUNK = E // CH


def _mm_body(ogx_ref, wt_ref, wrt_ref, prr_ref, x_ref, res_ref, ar_ref):
    xb = jnp.dot(ogx_ref[...], wt_ref[...], preferred_element_type=jnp.float32)
    x_ref[...] = xb
    res_ref[...] = jnp.dot(ogx_ref[...], wrt_ref[...],
                           preferred_element_type=jnp.float32)
    ar_ref[...] = jnp.dot(xb, prr_ref[...], preferred_element_type=jnp.float32)


def _tc_matmuls(og_x, wt, wrt, pr_rep):
    bm = 1000
    return pl.pallas_call(
        _mm_body,
        grid=(N // bm,),
        in_specs=[
            pl.BlockSpec((bm, D), lambda i: (i, 0)),
            pl.BlockSpec((D, D), lambda i: (0, 0)),
            pl.BlockSpec((D, D), lambda i: (0, 0)),
            pl.BlockSpec((D, D), lambda i: (0, 0)),
        ],
        out_specs=[
            pl.BlockSpec((bm, D), lambda i: (i, 0)),
            pl.BlockSpec((bm, D), lambda i: (i, 0)),
            pl.BlockSpec((bm, D), lambda i: (i, 0)),
        ],
        out_shape=[
            jax.ShapeDtypeStruct((N, D), jnp.float32),
            jax.ShapeDtypeStruct((N, D), jnp.float32),
            jax.ShapeDtypeStruct((N, D), jnp.float32),
        ],
    )(og_x, wt, wrt, pr_rep)


def _edge_body(xg_ref, arg_ref, ewb_ref, plr_ref, b8_ref, e_ref, emul_ref):
    alg = jnp.dot(xg_ref[...], plr_ref[...], preferred_element_type=jnp.float32)
    ew128 = jnp.dot(ewb_ref[...], b8_ref[...],
                    preferred_element_type=jnp.float32)
    z = ew128 * (alg + arg_ref[...])
    z = jnp.where(z >= 0.0, z, 0.2 * z)
    e128 = jnp.exp(z)
    e_ref[...] = e128
    emul_ref[...] = xg_ref[...] * e128


def _tc_edge(xg, arg, ewb, pl_rep, b8):
    bm = 2000
    return pl.pallas_call(
        _edge_body,
        grid=(E // bm,),
        in_specs=[
            pl.BlockSpec((bm, D), lambda i: (i, 0)),
            pl.BlockSpec((bm, D), lambda i: (i, 0)),
            pl.BlockSpec((bm, H), lambda i: (i, 0)),
            pl.BlockSpec((D, D), lambda i: (0, 0)),
            pl.BlockSpec((H, D), lambda i: (0, 0)),
        ],
        out_specs=[
            pl.BlockSpec((bm, D), lambda i: (i, 0)),
            pl.BlockSpec((bm, D), lambda i: (i, 0)),
        ],
        out_shape=[
            jax.ShapeDtypeStruct((E, D), jnp.float32),
            jax.ShapeDtypeStruct((E, D), jnp.float32),
        ],
    )(xg, arg, ewb, pl_rep, b8)


def _final_body(np_ref, dp_ref, res_ref, o_ref):
    den = jnp.sum(dp_ref[...], axis=0) + 1e-16
    o = jnp.sum(np_ref[...], axis=0) / den
    o = jnp.where(o > 0.0, o, jnp.exp(o) - 1.0)
    o_ref[...] = o + res_ref[...]


def _tc_final(numer_parts, denom_parts, res):
    ncp = numer_parts.shape[0]
    bm = 1000
    return pl.pallas_call(
        _final_body,
        grid=(N // bm,),
        in_specs=[
            pl.BlockSpec((ncp, bm, D), lambda i: (0, i, 0)),
            pl.BlockSpec((ncp, bm, D), lambda i: (0, i, 0)),
            pl.BlockSpec((bm, D), lambda i: (i, 0)),
        ],
        out_specs=pl.BlockSpec((bm, D), lambda i: (i, 0)),
        out_shape=jax.ShapeDtypeStruct((N, D), jnp.float32),
    )(numer_parts, denom_parts, res)


def _make_sc_gather(nc, ns):
    nw = nc * ns
    nloop = -(-NCHUNK // nw)
    mesh = plsc.VectorSubcoreMesh(core_axis_name="c", subcore_axis_name="s")

    @functools.partial(
        pl.kernel,
        mesh=mesh,
        out_type=[
            jax.ShapeDtypeStruct((E, D), jnp.float32),
            jax.ShapeDtypeStruct((E, D), jnp.float32),
        ],
        scratch_types=[
            pltpu.VMEM((CH,), jnp.int32),
            pltpu.VMEM((CH,), jnp.int32),
            pltpu.VMEM((CH, D), jnp.float32),
            pltpu.VMEM((CH, D), jnp.float32),
            pltpu.SemaphoreType.DMA,
        ],
    )
    def k(x_hbm, ar_hbm, src, dst, o_xg, o_arg, src_v, dst_v, xr_v, ar_v, sem):
        wid = lax.axis_index("s") * nc + lax.axis_index("c")

        def body(i, carry):
            cid = wid + i * nw

            @pl.when(cid < NCHUNK)
            def _():
                base = cid * CH
                pltpu.sync_copy(src.at[pl.ds(base, CH)], src_v)
                pltpu.sync_copy(dst.at[pl.ds(base, CH)], dst_v)
                c1 = pltpu.async_copy(x_hbm.at[src_v], xr_v, sem)
                c2 = pltpu.async_copy(ar_hbm.at[dst_v], ar_v, sem)
                c1.wait()
                c2.wait()
                pltpu.sync_copy(xr_v, o_xg.at[pl.ds(base, CH)])
                pltpu.sync_copy(ar_v, o_arg.at[pl.ds(base, CH)])

            return carry

        lax.fori_loop(0, nloop, body, 0)

    return k


def _make_sc_scatter(nc, ns):
    nw = nc * ns
    nloop = -(-NCHUNK // nw)
    mesh = plsc.VectorSubcoreMesh(core_axis_name="c", subcore_axis_name="s")

    @functools.partial(
        pl.kernel,
        mesh=mesh,
        out_type=[
            jax.ShapeDtypeStruct((nc, N, D), jnp.float32),
            jax.ShapeDtypeStruct((nc, N, D), jnp.float32),
        ],
        scratch_types=[
            pltpu.VMEM_SHARED((N, D), jnp.float32),
            pltpu.VMEM((CH,), jnp.int32),
            pltpu.VMEM((CH, D), jnp.float32),
        ],
    )
    def k(emul, e128, dst, zero_nd, o_num, o_den, shared, dst_v, val_v):
        cidx = lax.axis_index("c")
        sidx = lax.axis_index("s")
        wid = sidx * nc + cidx

        def scatter_phase(vals_hbm, out_hbm):
            @pl.when(sidx == 0)
            def _():
                pltpu.sync_copy(zero_nd, shared)

            plsc.subcore_barrier()

            def body(i, carry):
                cid = wid + i * nw

                @pl.when(cid < NCHUNK)
                def _():
                    base = cid * CH
                    pltpu.sync_copy(dst.at[pl.ds(base, CH)], dst_v)
                    pltpu.sync_copy(vals_hbm.at[pl.ds(base, CH)], val_v)
                    pltpu.sync_copy(val_v, shared.at[dst_v], add=True)

                return carry

            lax.fori_loop(0, nloop, body, 0)
            plsc.subcore_barrier()

            @pl.when(sidx == 0)
            def _():
                pltpu.sync_copy(shared, out_hbm.at[cidx])

            plsc.subcore_barrier()

        scatter_phase(emul, o_num)
        scatter_phase(e128, o_den)

    return k


def kernel(og_x, edge_index, edge_weight, W, att_l, att_r, W_res):
    src = edge_index[0]
    dst = edge_index[1]

    info = plsc.get_sparse_core_info()
    nc, ns = info.num_cores, info.num_subcores

    eye8 = jnp.eye(H, dtype=jnp.float32)
    pl_blk = (att_l[0][:, :, None] * eye8[:, None, :]).reshape(H * C, H)
    pr_blk = (att_r[0][:, :, None] * eye8[:, None, :]).reshape(H * C, H)
    b8 = jnp.repeat(eye8, C, axis=1)
    pl_rep = pl_blk @ b8
    pr_rep = pr_blk @ b8

    x, res, ar128 = _tc_matmuls(og_x, W.T, W_res.T, pr_rep)

    xg, arg = _make_sc_gather(nc, ns)(x, ar128, src, dst)

    ewb = jnp.repeat(edge_weight, H).reshape(E, H)
    e128, emul = _tc_edge(xg, arg, ewb, pl_rep, b8)

    zero_nd = jnp.zeros((N, D), jnp.float32)
    numer_parts, denom_parts = _make_sc_scatter(nc, ns)(
        emul, e128, dst, zero_nd)

    return _tc_final(numer_parts, denom_parts, res)
